# baseline (speedup 1.0000x reference)
import functools

import jax
import jax.numpy as jnp
from jax import lax
from jax.experimental import pallas as pl
from jax.experimental.pallas import tpu as pltpu
from jax.experimental.pallas import tpu_sc as plsc

_N = 10000
_E = 320000
_D = 128
_NACC = 10240
_CHUNK = 32
_NC = 2
_NS = 16
_NW = _NC * _NS
_G = 8
_NG = 40
_CPT = _G * _NG
_NB = 8
_RPN = _N // _NS
_EPAD = _NW * _CHUNK * _CPT
_RPT = _NACC // _NS


def _sc_agg_body(want_counts, table_hbm, srcs_hbm, dsts_hbm, *refs):
    if want_counts:
        out_hbm, cnt_hbm = refs[0], refs[1]
        scratch = refs[2:]
    else:
        out_hbm = refs[0]
        scratch = refs[1:]
    (src_ib, dst_ib, rows_v, ones_v, acc_sh, cnt_sh, sem_g, sem_i,
     sem_s) = scratch

    cid = lax.axis_index("c")
    sid = lax.axis_index("s")

    zeros16 = jnp.zeros((16,), jnp.float32)
    ones16 = jnp.ones((16,), jnp.float32)
    for k in range(_CHUNK // 16):
        ones_v[pl.ds(k * 16, 16)] = ones16

    @pl.loop(0, _CHUNK)
    def _(i):
        for k in range(_D // 16):
            rows_v[0, i, pl.ds(k * 16, 16)] = zeros16

    for b in range(_RPT // _CHUNK):
        pltpu.sync_copy(rows_v.at[0],
                        acc_sh.at[pl.ds(sid * _RPT + b * _CHUNK, _CHUNK)])
    if want_counts:
        for b in range(_RPT // _D):
            pltpu.sync_copy(rows_v.at[0, 0],
                            cnt_sh.at[pl.ds(sid * _RPT + b * _D, _D)])
    plsc.subcore_barrier()

    wid = sid * _NC + cid

    def _drain_gather(buf):
        pltpu.make_async_copy(table_hbm.at[src_ib.at[0, 0]], buf, sem_g).wait()

    def _drain_idx():
        pltpu.make_async_copy(srcs_hbm.at[wid, pl.ds(0, _G)], src_ib.at[0],
                              sem_i).wait()
        pltpu.make_async_copy(dsts_hbm.at[wid, pl.ds(0, _G)], dst_ib.at[0],
                              sem_i).wait()

    def _drain_scatter():
        pltpu.make_async_copy(rows_v.at[0],
                              acc_sh.at[src_ib.at[0, 0]], sem_s).wait()

    pltpu.sync_copy(srcs_hbm.at[wid, pl.ds(0, _G)], src_ib.at[0])
    pltpu.sync_copy(dsts_hbm.at[wid, pl.ds(0, _G)], dst_ib.at[0])
    for j in range(_NB - 1):
        pltpu.async_copy(table_hbm.at[src_ib.at[0, j]], rows_v.at[j], sem_g)

    @pl.loop(0, _NG)
    def _(g):
        b = g % 2
        for j in range(_G):
            _drain_gather(rows_v.at[j % _NB])
            if j == 0:
                @pl.when(g > 0)
                def _():
                    _drain_scatter()

                @pl.when(g + 1 < _NG)
                def _():
                    pltpu.async_copy(srcs_hbm.at[wid, pl.ds((g + 1) * _G, _G)],
                                     src_ib.at[(g + 1) % 2], sem_i)
                    pltpu.async_copy(dsts_hbm.at[wid, pl.ds((g + 1) * _G, _G)],
                                     dst_ib.at[(g + 1) % 2], sem_i)
            else:
                _drain_scatter()
            if j + _NB - 1 < _G:
                pltpu.async_copy(table_hbm.at[src_ib.at[b, j + _NB - 1]],
                                 rows_v.at[(j + _NB - 1) % _NB], sem_g)
            else:
                if j + _NB - 1 == _G:
                    @pl.when(g + 1 < _NG)
                    def _():
                        _drain_idx()
                        pltpu.async_copy(
                            table_hbm.at[src_ib.at[(g + 1) % 2, 0]],
                            rows_v.at[(j + _NB - 1) % _NB], sem_g)
                else:
                    @pl.when(g + 1 < _NG)
                    def _():
                        pltpu.async_copy(
                            table_hbm.at[src_ib.at[(g + 1) % 2, j + _NB - 1 - _G]],
                            rows_v.at[(j + _NB - 1) % _NB], sem_g)
            pltpu.async_copy(rows_v.at[j % _NB], acc_sh.at[dst_ib.at[b, j]],
                             sem_s, add=True)
            if want_counts:
                pltpu.sync_copy(ones_v, cnt_sh.at[dst_ib.at[b, j]], add=True)

    _drain_scatter()
    plsc.subcore_barrier()

    pltpu.sync_copy(acc_sh.at[pl.ds(sid * _RPT, _RPT)],
                    out_hbm.at[cid, pl.ds(sid * _RPT, _RPT)])
    if want_counts:
        pltpu.sync_copy(cnt_sh.at[pl.ds(sid * _RPT, _RPT)],
                        cnt_hbm.at[cid, pl.ds(sid * _RPT, _RPT)])


def _make_sc_agg(want_counts):
    mesh = plsc.VectorSubcoreMesh(core_axis_name="c", subcore_axis_name="s",
                                  num_cores=_NC, num_subcores=_NS)
    out_type = [jax.ShapeDtypeStruct((_NC, _NACC, _D), jnp.float32)]
    if want_counts:
        out_type.append(jax.ShapeDtypeStruct((_NC, _NACC), jnp.float32))
    scratch = [
        pltpu.VMEM((2, _G, _CHUNK), jnp.int32),
        pltpu.VMEM((2, _G, _CHUNK), jnp.int32),
        pltpu.VMEM((_NB, _CHUNK, _D), jnp.float32),
        pltpu.VMEM((_CHUNK,), jnp.float32),
        pltpu.VMEM_SHARED((_NACC, _D), jnp.float32),
        pltpu.VMEM_SHARED((_NACC,), jnp.float32),
        pltpu.SemaphoreType.DMA,
        pltpu.SemaphoreType.DMA,
        pltpu.SemaphoreType.DMA,
    ]
    return pl.kernel(
        functools.partial(_sc_agg_body, want_counts),
        out_type=out_type if want_counts else out_type[0],
        mesh=mesh,
        scratch_types=scratch,
    )


_BLK = 400
_GRID = _N // _BLK


def _tc1_body(p_ref, c_ref, x_ref, w1l_ref, b1_ref, w1r_ref, w2l_ref,
              w2r_ref, b2_ref, t2_ref, r2_ref):
    cnt = c_ref[0, :, 0] + c_ref[1, :, 0]
    inv = 1.0 / jnp.maximum(cnt, 1.0)
    agg = (p_ref[0] + p_ref[1]) * inv[:, None]
    h1 = agg @ w1l_ref[...] + b1_ref[...] + x_ref[...] @ w1r_ref[...]
    h1 = jnp.maximum(h1, 0.0)
    t2_ref[...] = h1 @ w2l_ref[...]
    r2_ref[...] = h1 @ w2r_ref[...] + b2_ref[...]


def _tc1(P, C, xpad, W1l, b1, W1r, W2l, W2r, b2):
    return pl.pallas_call(
        _tc1_body,
        grid=(_GRID,),
        in_specs=[
            pl.BlockSpec((2, _BLK, _D), lambda i: (0, i, 0)),
            pl.BlockSpec((2, _BLK, 1), lambda i: (0, i, 0)),
            pl.BlockSpec((_BLK, _D), lambda i: (i, 0)),
            pl.BlockSpec((_D, 256), lambda i: (0, 0)),
            pl.BlockSpec((1, 256), lambda i: (0, 0)),
            pl.BlockSpec((_D, 256), lambda i: (0, 0)),
            pl.BlockSpec((256, _D), lambda i: (0, 0)),
            pl.BlockSpec((256, _D), lambda i: (0, 0)),
            pl.BlockSpec((1, _D), lambda i: (0, 0)),
        ],
        out_specs=[
            pl.BlockSpec((_BLK, _D), lambda i: (i, 0)),
            pl.BlockSpec((_BLK, _D), lambda i: (i, 0)),
        ],
        out_shape=[
            jax.ShapeDtypeStruct((_N, _D), jnp.float32),
            jax.ShapeDtypeStruct((_N, _D), jnp.float32),
        ],
    )(P, C, xpad, W1l, b1, W1r, W2l, W2r, b2)


def _tc2_body(s_ref, c_ref, r_ref, wl_ref, wr_ref, b_ref, t_ref, rout_ref):
    cnt = c_ref[0, :, 0] + c_ref[1, :, 0]
    inv = 1.0 / jnp.maximum(cnt, 1.0)
    h = (s_ref[0] + s_ref[1]) * inv[:, None] + r_ref[...]
    h = jnp.maximum(h, 0.0)
    t_ref[...] = h @ wl_ref[...]
    rout_ref[...] = h @ wr_ref[...] + b_ref[...]


def _tc2(S, C, r, Wl, Wr, b):
    return pl.pallas_call(
        _tc2_body,
        grid=(_GRID,),
        in_specs=[
            pl.BlockSpec((2, _BLK, _D), lambda i: (0, i, 0)),
            pl.BlockSpec((2, _BLK, 1), lambda i: (0, i, 0)),
            pl.BlockSpec((_BLK, _D), lambda i: (i, 0)),
            pl.BlockSpec((_D, _D), lambda i: (0, 0)),
            pl.BlockSpec((_D, _D), lambda i: (0, 0)),
            pl.BlockSpec((1, _D), lambda i: (0, 0)),
        ],
        out_specs=[
            pl.BlockSpec((_BLK, _D), lambda i: (i, 0)),
            pl.BlockSpec((_BLK, _D), lambda i: (i, 0)),
        ],
        out_shape=[
            jax.ShapeDtypeStruct((_N, _D), jnp.float32),
            jax.ShapeDtypeStruct((_N, _D), jnp.float32),
        ],
    )(S, C, r, Wl, Wr, b)


def _tc3_body(s_ref, c_ref, r_ref, o_ref):
    cnt = c_ref[0, :, 0] + c_ref[1, :, 0]
    inv = 1.0 / jnp.maximum(cnt, 1.0)
    h = (s_ref[0] + s_ref[1]) * inv[:, None] + r_ref[...]
    o_ref[...] = jax.nn.sigmoid(h)


def _tc3(S, C, r):
    return pl.pallas_call(
        _tc3_body,
        grid=(_GRID,),
        in_specs=[
            pl.BlockSpec((2, _BLK, _D), lambda i: (0, i, 0)),
            pl.BlockSpec((2, _BLK, 1), lambda i: (0, i, 0)),
            pl.BlockSpec((_BLK, _D), lambda i: (i, 0)),
        ],
        out_specs=pl.BlockSpec((_BLK, _D), lambda i: (i, 0)),
        out_shape=jax.ShapeDtypeStruct((_N, _D), jnp.float32),
    )(S, C, r)


def kernel(x, edge_index, W1l, b1, W1r, W2l, b2, W2r, W3l, b3, W3r):
    src = edge_index[0]
    dst = edge_index[1]
    npad = _EPAD - _E
    pad_i = jnp.arange(npad, dtype=jnp.int32)
    pad_src = (pad_i * 97) % _N
    pad_dst = _N + pad_i % (_NACC - _N)
    srcs = jnp.concatenate([src, pad_src]).reshape(_NW, _CPT, _CHUNK)
    dsts = jnp.concatenate([dst, pad_dst]).reshape(_NW, _CPT, _CHUNK)

    S1, Craw = _make_sc_agg(True)(x, srcs, dsts)
    C = Craw[:, :_N].reshape(_NC, _N, 1)
    t2, r2 = _tc1(S1, C, x, W1l, b1.reshape(1, -1), W1r, W2l, W2r,
                  b2.reshape(1, -1))
    S2 = _make_sc_agg(False)(t2, srcs, dsts)
    t3, r3 = _tc2(S2, C, r2, W3l, W3r, b3.reshape(1, -1))
    S3 = _make_sc_agg(False)(t3, srcs, dsts)
    return _tc3(S3, C, r3)


# speedup vs baseline: 1.1343x; 1.1343x over previous
import functools

import jax
import jax.numpy as jnp
from jax import lax
from jax.experimental import pallas as pl
from jax.experimental.pallas import tpu as pltpu
from jax.experimental.pallas import tpu_sc as plsc

_N = 10000
_E = 320000
_D = 128
_NACC = 10240
_CHUNK = 64
_NC = 2
_NS = 16
_NW = _NC * _NS
_G = 8
_NG = 20
_CPT = _G * _NG
_NB = 4
_RPN = _N // _NS
_EPAD = _NW * _CHUNK * _CPT
_RPT = _NACC // _NS


def _sc_agg_body(want_counts, table_hbm, srcs_hbm, dsts_hbm, *refs):
    if want_counts:
        out_hbm, cnt_hbm = refs[0], refs[1]
        scratch = refs[2:]
    else:
        out_hbm = refs[0]
        scratch = refs[1:]
    (src_ib, dst_ib, rows_v, ones_v, acc_sh, cnt_sh, sem_g, sem_i,
     sem_s) = scratch

    cid = lax.axis_index("c")
    sid = lax.axis_index("s")

    zeros16 = jnp.zeros((16,), jnp.float32)
    ones16 = jnp.ones((16,), jnp.float32)
    for k in range(_CHUNK // 16):
        ones_v[pl.ds(k * 16, 16)] = ones16

    @pl.loop(0, _CHUNK)
    def _(i):
        for k in range(_D // 16):
            rows_v[0, i, pl.ds(k * 16, 16)] = zeros16

    for b in range(_RPT // _CHUNK):
        pltpu.sync_copy(rows_v.at[0],
                        acc_sh.at[pl.ds(sid * _RPT + b * _CHUNK, _CHUNK)])
    if want_counts:
        for b in range(_RPT // _D):
            pltpu.sync_copy(rows_v.at[0, 0],
                            cnt_sh.at[pl.ds(sid * _RPT + b * _D, _D)])
    plsc.subcore_barrier()

    wid = sid * _NC + cid

    def _drain_gather(buf):
        pltpu.make_async_copy(table_hbm.at[src_ib.at[0, 0]], buf, sem_g).wait()

    def _drain_idx():
        pltpu.make_async_copy(srcs_hbm.at[wid, pl.ds(0, _G)], src_ib.at[0],
                              sem_i).wait()
        pltpu.make_async_copy(dsts_hbm.at[wid, pl.ds(0, _G)], dst_ib.at[0],
                              sem_i).wait()

    def _drain_scatter():
        pltpu.make_async_copy(rows_v.at[0],
                              acc_sh.at[src_ib.at[0, 0]], sem_s).wait()

    pltpu.sync_copy(srcs_hbm.at[wid, pl.ds(0, _G)], src_ib.at[0])
    pltpu.sync_copy(dsts_hbm.at[wid, pl.ds(0, _G)], dst_ib.at[0])
    for j in range(_NB - 1):
        pltpu.async_copy(table_hbm.at[src_ib.at[0, j]], rows_v.at[j], sem_g)

    @pl.loop(0, _NG)
    def _(g):
        b = g % 2
        for j in range(_G):
            _drain_gather(rows_v.at[j % _NB])
            if j == 0:
                @pl.when(g > 0)
                def _():
                    _drain_scatter()

                @pl.when(g + 1 < _NG)
                def _():
                    pltpu.async_copy(srcs_hbm.at[wid, pl.ds((g + 1) * _G, _G)],
                                     src_ib.at[(g + 1) % 2], sem_i)
                    pltpu.async_copy(dsts_hbm.at[wid, pl.ds((g + 1) * _G, _G)],
                                     dst_ib.at[(g + 1) % 2], sem_i)
            else:
                _drain_scatter()
            if j + _NB - 1 < _G:
                pltpu.async_copy(table_hbm.at[src_ib.at[b, j + _NB - 1]],
                                 rows_v.at[(j + _NB - 1) % _NB], sem_g)
            else:
                if j + _NB - 1 == _G:
                    @pl.when(g + 1 < _NG)
                    def _():
                        _drain_idx()
                        pltpu.async_copy(
                            table_hbm.at[src_ib.at[(g + 1) % 2, 0]],
                            rows_v.at[(j + _NB - 1) % _NB], sem_g)
                else:
                    @pl.when(g + 1 < _NG)
                    def _():
                        pltpu.async_copy(
                            table_hbm.at[src_ib.at[(g + 1) % 2, j + _NB - 1 - _G]],
                            rows_v.at[(j + _NB - 1) % _NB], sem_g)
            pltpu.async_copy(rows_v.at[j % _NB], acc_sh.at[dst_ib.at[b, j]],
                             sem_s, add=True)
            if want_counts:
                pltpu.sync_copy(ones_v, cnt_sh.at[dst_ib.at[b, j]], add=True)

    _drain_scatter()
    plsc.subcore_barrier()

    pltpu.sync_copy(acc_sh.at[pl.ds(sid * _RPT, _RPT)],
                    out_hbm.at[cid, pl.ds(sid * _RPT, _RPT)])
    if want_counts:
        pltpu.sync_copy(cnt_sh.at[pl.ds(sid * _RPT, _RPT)],
                        cnt_hbm.at[cid, pl.ds(sid * _RPT, _RPT)])


def _make_sc_agg(want_counts):
    mesh = plsc.VectorSubcoreMesh(core_axis_name="c", subcore_axis_name="s",
                                  num_cores=_NC, num_subcores=_NS)
    out_type = [jax.ShapeDtypeStruct((_NC, _NACC, _D), jnp.float32)]
    if want_counts:
        out_type.append(jax.ShapeDtypeStruct((_NC, _NACC), jnp.float32))
    scratch = [
        pltpu.VMEM((2, _G, _CHUNK), jnp.int32),
        pltpu.VMEM((2, _G, _CHUNK), jnp.int32),
        pltpu.VMEM((_NB, _CHUNK, _D), jnp.float32),
        pltpu.VMEM((_CHUNK,), jnp.float32),
        pltpu.VMEM_SHARED((_NACC, _D), jnp.float32),
        pltpu.VMEM_SHARED((_NACC,), jnp.float32),
        pltpu.SemaphoreType.DMA,
        pltpu.SemaphoreType.DMA,
        pltpu.SemaphoreType.DMA,
    ]
    return pl.kernel(
        functools.partial(_sc_agg_body, want_counts),
        out_type=out_type if want_counts else out_type[0],
        mesh=mesh,
        scratch_types=scratch,
    )


_BLK = 400
_GRID = _N // _BLK


def _tc1_body(p_ref, c_ref, x_ref, w1l_ref, b1_ref, w1r_ref, w2l_ref,
              w2r_ref, b2_ref, t2_ref, r2_ref):
    cnt = c_ref[0, :, 0] + c_ref[1, :, 0]
    inv = 1.0 / jnp.maximum(cnt, 1.0)
    agg = (p_ref[0] + p_ref[1]) * inv[:, None]
    h1 = agg @ w1l_ref[...] + b1_ref[...] + x_ref[...] @ w1r_ref[...]
    h1 = jnp.maximum(h1, 0.0)
    t2_ref[...] = h1 @ w2l_ref[...]
    r2_ref[...] = h1 @ w2r_ref[...] + b2_ref[...]


def _tc1(P, C, xpad, W1l, b1, W1r, W2l, W2r, b2):
    return pl.pallas_call(
        _tc1_body,
        grid=(_GRID,),
        in_specs=[
            pl.BlockSpec((2, _BLK, _D), lambda i: (0, i, 0)),
            pl.BlockSpec((2, _BLK, 1), lambda i: (0, i, 0)),
            pl.BlockSpec((_BLK, _D), lambda i: (i, 0)),
            pl.BlockSpec((_D, 256), lambda i: (0, 0)),
            pl.BlockSpec((1, 256), lambda i: (0, 0)),
            pl.BlockSpec((_D, 256), lambda i: (0, 0)),
            pl.BlockSpec((256, _D), lambda i: (0, 0)),
            pl.BlockSpec((256, _D), lambda i: (0, 0)),
            pl.BlockSpec((1, _D), lambda i: (0, 0)),
        ],
        out_specs=[
            pl.BlockSpec((_BLK, _D), lambda i: (i, 0)),
            pl.BlockSpec((_BLK, _D), lambda i: (i, 0)),
        ],
        out_shape=[
            jax.ShapeDtypeStruct((_N, _D), jnp.float32),
            jax.ShapeDtypeStruct((_N, _D), jnp.float32),
        ],
    )(P, C, xpad, W1l, b1, W1r, W2l, W2r, b2)


def _tc2_body(s_ref, c_ref, r_ref, wl_ref, wr_ref, b_ref, t_ref, rout_ref):
    cnt = c_ref[0, :, 0] + c_ref[1, :, 0]
    inv = 1.0 / jnp.maximum(cnt, 1.0)
    h = (s_ref[0] + s_ref[1]) * inv[:, None] + r_ref[...]
    h = jnp.maximum(h, 0.0)
    t_ref[...] = h @ wl_ref[...]
    rout_ref[...] = h @ wr_ref[...] + b_ref[...]


def _tc2(S, C, r, Wl, Wr, b):
    return pl.pallas_call(
        _tc2_body,
        grid=(_GRID,),
        in_specs=[
            pl.BlockSpec((2, _BLK, _D), lambda i: (0, i, 0)),
            pl.BlockSpec((2, _BLK, 1), lambda i: (0, i, 0)),
            pl.BlockSpec((_BLK, _D), lambda i: (i, 0)),
            pl.BlockSpec((_D, _D), lambda i: (0, 0)),
            pl.BlockSpec((_D, _D), lambda i: (0, 0)),
            pl.BlockSpec((1, _D), lambda i: (0, 0)),
        ],
        out_specs=[
            pl.BlockSpec((_BLK, _D), lambda i: (i, 0)),
            pl.BlockSpec((_BLK, _D), lambda i: (i, 0)),
        ],
        out_shape=[
            jax.ShapeDtypeStruct((_N, _D), jnp.float32),
            jax.ShapeDtypeStruct((_N, _D), jnp.float32),
        ],
    )(S, C, r, Wl, Wr, b)


def _tc3_body(s_ref, c_ref, r_ref, o_ref):
    cnt = c_ref[0, :, 0] + c_ref[1, :, 0]
    inv = 1.0 / jnp.maximum(cnt, 1.0)
    h = (s_ref[0] + s_ref[1]) * inv[:, None] + r_ref[...]
    o_ref[...] = jax.nn.sigmoid(h)


def _tc3(S, C, r):
    return pl.pallas_call(
        _tc3_body,
        grid=(_GRID,),
        in_specs=[
            pl.BlockSpec((2, _BLK, _D), lambda i: (0, i, 0)),
            pl.BlockSpec((2, _BLK, 1), lambda i: (0, i, 0)),
            pl.BlockSpec((_BLK, _D), lambda i: (i, 0)),
        ],
        out_specs=pl.BlockSpec((_BLK, _D), lambda i: (i, 0)),
        out_shape=jax.ShapeDtypeStruct((_N, _D), jnp.float32),
    )(S, C, r)


def kernel(x, edge_index, W1l, b1, W1r, W2l, b2, W2r, W3l, b3, W3r):
    src = edge_index[0]
    dst = edge_index[1]
    npad = _EPAD - _E
    pad_i = jnp.arange(npad, dtype=jnp.int32)
    pad_src = (pad_i * 97) % _N
    pad_dst = _N + pad_i % (_NACC - _N)
    srcs = jnp.concatenate([src, pad_src]).reshape(_NW, _CPT, _CHUNK)
    dsts = jnp.concatenate([dst, pad_dst]).reshape(_NW, _CPT, _CHUNK)

    S1, Craw = _make_sc_agg(True)(x, srcs, dsts)
    C = Craw[:, :_N].reshape(_NC, _N, 1)
    t2, r2 = _tc1(S1, C, x, W1l, b1.reshape(1, -1), W1r, W2l, W2r,
                  b2.reshape(1, -1))
    S2 = _make_sc_agg(False)(t2, srcs, dsts)
    t3, r3 = _tc2(S2, C, r2, W3l, W3r, b3.reshape(1, -1))
    S3 = _make_sc_agg(False)(t3, srcs, dsts)
    return _tc3(S3, C, r3)


# speedup vs baseline: 1.2499x; 1.1019x over previous
import functools

import jax
import jax.numpy as jnp
from jax import lax
from jax.experimental import pallas as pl
from jax.experimental.pallas import tpu as pltpu
from jax.experimental.pallas import tpu_sc as plsc

_N = 10000
_E = 320000
_D = 128
_NACC = 10240
_CHUNK = 64
_NC = 2
_NS = 16
_NW = _NC * _NS
_G = 8
_NG = 20
_CPT = _G * _NG
_NB = 4
_RPN = _N // _NS
_EPAD = _NW * _CHUNK * _CPT
_RPT = _NACC // _NS


def _sc_agg_body(want_counts, table_hbm, edges_hbm, *refs):
    if want_counts:
        out_hbm, cnt_hbm = refs[0], refs[1]
        scratch = refs[2:]
    else:
        out_hbm = refs[0]
        scratch = refs[1:]
    (src_ib, dst_ib, rows_v, ones_v, acc_sh, cnt_sh, sem_g, sem_i,
     sem_s) = scratch

    cid = lax.axis_index("c")
    sid = lax.axis_index("s")

    zeros16 = jnp.zeros((16,), jnp.float32)
    ones16 = jnp.ones((16,), jnp.float32)
    for k in range(_CHUNK // 16):
        ones_v[pl.ds(k * 16, 16)] = ones16

    @pl.loop(0, _CHUNK)
    def _(i):
        for k in range(_D // 16):
            rows_v[0, i, pl.ds(k * 16, 16)] = zeros16

    for b in range(_RPT // _CHUNK):
        pltpu.sync_copy(rows_v.at[0],
                        acc_sh.at[pl.ds(sid * _RPT + b * _CHUNK, _CHUNK)])
    if want_counts:
        for b in range(_RPT // _D):
            pltpu.sync_copy(rows_v.at[0, 0],
                            cnt_sh.at[pl.ds(sid * _RPT + b * _D, _D)])
    plsc.subcore_barrier()

    wid = sid * _NC + cid

    def _drain_gather(buf):
        pltpu.make_async_copy(table_hbm.at[src_ib.at[0, 0]], buf, sem_g).wait()

    def _drain_idx():
        pltpu.make_async_copy(edges_hbm.at[0, wid, pl.ds(0, _G)], src_ib.at[0],
                              sem_i).wait()
        pltpu.make_async_copy(edges_hbm.at[1, wid, pl.ds(0, _G)], dst_ib.at[0],
                              sem_i).wait()

    def _drain_scatter():
        pltpu.make_async_copy(rows_v.at[0],
                              acc_sh.at[src_ib.at[0, 0]], sem_s).wait()

    pltpu.sync_copy(edges_hbm.at[0, wid, pl.ds(0, _G)], src_ib.at[0])
    pltpu.sync_copy(edges_hbm.at[1, wid, pl.ds(0, _G)], dst_ib.at[0])
    for j in range(_NB - 1):
        pltpu.async_copy(table_hbm.at[src_ib.at[0, j]], rows_v.at[j], sem_g)

    @pl.loop(0, _NG)
    def _(g):
        b = g % 2
        for j in range(_G):
            _drain_gather(rows_v.at[j % _NB])
            if j == 0:
                @pl.when(g > 0)
                def _():
                    _drain_scatter()

                @pl.when(g + 1 < _NG)
                def _():
                    pltpu.async_copy(
                        edges_hbm.at[0, wid, pl.ds((g + 1) * _G, _G)],
                        src_ib.at[(g + 1) % 2], sem_i)
                    pltpu.async_copy(
                        edges_hbm.at[1, wid, pl.ds((g + 1) * _G, _G)],
                        dst_ib.at[(g + 1) % 2], sem_i)
            else:
                _drain_scatter()
            if j + _NB - 1 < _G:
                pltpu.async_copy(table_hbm.at[src_ib.at[b, j + _NB - 1]],
                                 rows_v.at[(j + _NB - 1) % _NB], sem_g)
            else:
                if j + _NB - 1 == _G:
                    @pl.when(g + 1 < _NG)
                    def _():
                        _drain_idx()
                        pltpu.async_copy(
                            table_hbm.at[src_ib.at[(g + 1) % 2, 0]],
                            rows_v.at[(j + _NB - 1) % _NB], sem_g)
                else:
                    @pl.when(g + 1 < _NG)
                    def _():
                        pltpu.async_copy(
                            table_hbm.at[src_ib.at[(g + 1) % 2, j + _NB - 1 - _G]],
                            rows_v.at[(j + _NB - 1) % _NB], sem_g)
            pltpu.async_copy(rows_v.at[j % _NB], acc_sh.at[dst_ib.at[b, j]],
                             sem_s, add=True)
            if want_counts:
                pltpu.sync_copy(ones_v, cnt_sh.at[dst_ib.at[b, j]], add=True)

    _drain_scatter()
    plsc.subcore_barrier()

    pltpu.sync_copy(acc_sh.at[pl.ds(sid * _RPT, _RPT)],
                    out_hbm.at[cid, pl.ds(sid * _RPT, _RPT)])
    if want_counts:
        pltpu.sync_copy(cnt_sh.at[pl.ds(sid * _RPT, _RPT)],
                        cnt_hbm.at[cid, pl.ds(sid * _RPT, _RPT)])


def _make_sc_agg(want_counts):
    mesh = plsc.VectorSubcoreMesh(core_axis_name="c", subcore_axis_name="s",
                                  num_cores=_NC, num_subcores=_NS)
    out_type = [jax.ShapeDtypeStruct((_NC, _NACC, _D), jnp.float32)]
    if want_counts:
        out_type.append(jax.ShapeDtypeStruct((_NC, _NACC), jnp.float32))
    scratch = [
        pltpu.VMEM((2, _G, _CHUNK), jnp.int32),
        pltpu.VMEM((2, _G, _CHUNK), jnp.int32),
        pltpu.VMEM((_NB, _CHUNK, _D), jnp.float32),
        pltpu.VMEM((_CHUNK,), jnp.float32),
        pltpu.VMEM_SHARED((_NACC, _D), jnp.float32),
        pltpu.VMEM_SHARED((_NACC,), jnp.float32),
        pltpu.SemaphoreType.DMA,
        pltpu.SemaphoreType.DMA,
        pltpu.SemaphoreType.DMA,
    ]
    return pl.kernel(
        functools.partial(_sc_agg_body, want_counts),
        out_type=out_type if want_counts else out_type[0],
        mesh=mesh,
        scratch_types=scratch,
    )


_BLK = 1000
_GRID = _N // _BLK


def _tc1_body(p_ref, i_ref, x_ref, w1l_ref, b1_ref, w1r_ref, w2l_ref,
              w2r_ref, b2_ref, t2_ref, r2_ref):
    agg = (p_ref[0] + p_ref[1]) * i_ref[...]
    h1 = agg @ w1l_ref[...] + b1_ref[...] + x_ref[...] @ w1r_ref[...]
    h1 = jnp.maximum(h1, 0.0)
    t2_ref[...] = h1 @ w2l_ref[...]
    r2_ref[...] = h1 @ w2r_ref[...] + b2_ref[...]


def _tc1(P, C, xpad, W1l, b1, W1r, W2l, W2r, b2):
    return pl.pallas_call(
        _tc1_body,
        grid=(_GRID,),
        in_specs=[
            pl.BlockSpec((2, _BLK, _D), lambda i: (0, i, 0)),
            pl.BlockSpec((_BLK, 1), lambda i: (i, 0)),
            pl.BlockSpec((_BLK, _D), lambda i: (i, 0)),
            pl.BlockSpec((_D, 256), lambda i: (0, 0)),
            pl.BlockSpec((1, 256), lambda i: (0, 0)),
            pl.BlockSpec((_D, 256), lambda i: (0, 0)),
            pl.BlockSpec((256, _D), lambda i: (0, 0)),
            pl.BlockSpec((256, _D), lambda i: (0, 0)),
            pl.BlockSpec((1, _D), lambda i: (0, 0)),
        ],
        out_specs=[
            pl.BlockSpec((_BLK, _D), lambda i: (i, 0)),
            pl.BlockSpec((_BLK, _D), lambda i: (i, 0)),
        ],
        out_shape=[
            jax.ShapeDtypeStruct((_N, _D), jnp.float32),
            jax.ShapeDtypeStruct((_N, _D), jnp.float32),
        ],
    )(P, C, xpad, W1l, b1, W1r, W2l, W2r, b2)


def _tc2_body(s_ref, i_ref, r_ref, wl_ref, wr_ref, b_ref, t_ref, rout_ref):
    h = (s_ref[0] + s_ref[1]) * i_ref[...] + r_ref[...]
    h = jnp.maximum(h, 0.0)
    t_ref[...] = h @ wl_ref[...]
    rout_ref[...] = h @ wr_ref[...] + b_ref[...]


def _tc2(S, C, r, Wl, Wr, b):
    return pl.pallas_call(
        _tc2_body,
        grid=(_GRID,),
        in_specs=[
            pl.BlockSpec((2, _BLK, _D), lambda i: (0, i, 0)),
            pl.BlockSpec((_BLK, 1), lambda i: (i, 0)),
            pl.BlockSpec((_BLK, _D), lambda i: (i, 0)),
            pl.BlockSpec((_D, _D), lambda i: (0, 0)),
            pl.BlockSpec((_D, _D), lambda i: (0, 0)),
            pl.BlockSpec((1, _D), lambda i: (0, 0)),
        ],
        out_specs=[
            pl.BlockSpec((_BLK, _D), lambda i: (i, 0)),
            pl.BlockSpec((_BLK, _D), lambda i: (i, 0)),
        ],
        out_shape=[
            jax.ShapeDtypeStruct((_N, _D), jnp.float32),
            jax.ShapeDtypeStruct((_N, _D), jnp.float32),
        ],
    )(S, C, r, Wl, Wr, b)


def _tc3_body(s_ref, i_ref, r_ref, o_ref):
    h = (s_ref[0] + s_ref[1]) * i_ref[...] + r_ref[...]
    o_ref[...] = jax.nn.sigmoid(h)


def _tc3(S, C, r):
    return pl.pallas_call(
        _tc3_body,
        grid=(_GRID,),
        in_specs=[
            pl.BlockSpec((2, _BLK, _D), lambda i: (0, i, 0)),
            pl.BlockSpec((_BLK, 1), lambda i: (i, 0)),
            pl.BlockSpec((_BLK, _D), lambda i: (i, 0)),
        ],
        out_specs=pl.BlockSpec((_BLK, _D), lambda i: (i, 0)),
        out_shape=jax.ShapeDtypeStruct((_N, _D), jnp.float32),
    )(S, C, r)


def kernel(x, edge_index, W1l, b1, W1r, W2l, b2, W2r, W3l, b3, W3r):
    npad = _EPAD - _E
    pad_i = jnp.arange(npad, dtype=jnp.int32)
    pads = jnp.stack([(pad_i * 97) % _N, _N + pad_i % (_NACC - _N)])
    edges = jnp.concatenate([edge_index, pads], axis=1)
    edges = edges.reshape(2, _NW, _CPT, _CHUNK)

    S1, Craw = _make_sc_agg(True)(x, edges)
    inv = (1.0 / jnp.maximum(Craw[0, :_N] + Craw[1, :_N], 1.0))[:, None]
    t2, r2 = _tc1(S1, inv, x, W1l, b1.reshape(1, -1), W1r, W2l, W2r,
                  b2.reshape(1, -1))
    S2 = _make_sc_agg(False)(t2, edges)
    t3, r3 = _tc2(S2, inv, r2, W3l, W3r, b3.reshape(1, -1))
    S3 = _make_sc_agg(False)(t3, edges)
    return _tc3(S3, inv, r3)


# speedup vs baseline: 1.2765x; 1.0213x over previous
import functools

import jax
import jax.numpy as jnp
from jax import lax
from jax.experimental import pallas as pl
from jax.experimental.pallas import tpu as pltpu
from jax.experimental.pallas import tpu_sc as plsc

_N = 10000
_E = 320000
_D = 128
_NACC = 10240
_CHUNK = 64
_NC = 2
_NS = 16
_NW = _NC * _NS
_G = 8
_NG = 20
_CPT = _G * _NG
_NB = 4
_RPN = _N // _NS
_EPAD = _NW * _CHUNK * _CPT
_RPT = _NACC // _NS


def _sc_agg_body(want_counts, table_hbm, edges_hbm, *refs):
    if want_counts:
        out_hbm, cnt_hbm = refs[0], refs[1]
        scratch = refs[2:]
    else:
        out_hbm = refs[0]
        scratch = refs[1:]
    (src_ib, dst_ib, rows_v, ones_v, acc_sh, cnt_sh, sem_g, sem_i,
     sem_s) = scratch

    cid = lax.axis_index("c")
    sid = lax.axis_index("s")

    zeros16 = jnp.zeros((16,), jnp.float32)
    ones16 = jnp.ones((16,), jnp.float32)
    for k in range(_CHUNK // 16):
        ones_v[pl.ds(k * 16, 16)] = ones16

    @pl.loop(0, _CHUNK)
    def _(i):
        for k in range(_D // 16):
            rows_v[0, i, pl.ds(k * 16, 16)] = zeros16

    for b in range(_RPT // _CHUNK):
        pltpu.sync_copy(rows_v.at[0],
                        acc_sh.at[pl.ds(sid * _RPT + b * _CHUNK, _CHUNK)])
    if want_counts:
        for b in range(_RPT // _D):
            pltpu.sync_copy(rows_v.at[0, 0],
                            cnt_sh.at[pl.ds(sid * _RPT + b * _D, _D)])
    plsc.subcore_barrier()

    wid = sid * _NC + cid

    def _drain_gather(buf):
        pltpu.make_async_copy(table_hbm.at[src_ib.at[0, 0]], buf, sem_g).wait()

    def _drain_idx():
        pltpu.make_async_copy(edges_hbm.at[0, wid, pl.ds(0, _G)], src_ib.at[0],
                              sem_i).wait()
        pltpu.make_async_copy(edges_hbm.at[1, wid, pl.ds(0, _G)], dst_ib.at[0],
                              sem_i).wait()

    def _drain_scatter():
        pltpu.make_async_copy(rows_v.at[0],
                              acc_sh.at[src_ib.at[0, 0]], sem_s).wait()

    pltpu.sync_copy(edges_hbm.at[0, wid, pl.ds(0, _G)], src_ib.at[0])
    pltpu.sync_copy(edges_hbm.at[1, wid, pl.ds(0, _G)], dst_ib.at[0])
    for j in range(_NB - 1):
        pltpu.async_copy(table_hbm.at[src_ib.at[0, j]], rows_v.at[j], sem_g)

    @pl.loop(0, _NG)
    def _(g):
        b = g % 2
        for j in range(_G):
            _drain_gather(rows_v.at[j % _NB])
            if j == 0:
                @pl.when(g > 0)
                def _():
                    _drain_scatter()

                @pl.when(g + 1 < _NG)
                def _():
                    pltpu.async_copy(
                        edges_hbm.at[0, wid, pl.ds((g + 1) * _G, _G)],
                        src_ib.at[(g + 1) % 2], sem_i)
                    pltpu.async_copy(
                        edges_hbm.at[1, wid, pl.ds((g + 1) * _G, _G)],
                        dst_ib.at[(g + 1) % 2], sem_i)
            else:
                _drain_scatter()
            if j + _NB - 1 < _G:
                pltpu.async_copy(table_hbm.at[src_ib.at[b, j + _NB - 1]],
                                 rows_v.at[(j + _NB - 1) % _NB], sem_g)
            else:
                if j + _NB - 1 == _G:
                    @pl.when(g + 1 < _NG)
                    def _():
                        _drain_idx()
                        pltpu.async_copy(
                            table_hbm.at[src_ib.at[(g + 1) % 2, 0]],
                            rows_v.at[(j + _NB - 1) % _NB], sem_g)
                else:
                    @pl.when(g + 1 < _NG)
                    def _():
                        pltpu.async_copy(
                            table_hbm.at[src_ib.at[(g + 1) % 2, j + _NB - 1 - _G]],
                            rows_v.at[(j + _NB - 1) % _NB], sem_g)
            pltpu.async_copy(rows_v.at[j % _NB], acc_sh.at[dst_ib.at[b, j]],
                             sem_s, add=True)
            if want_counts:
                pltpu.sync_copy(ones_v, cnt_sh.at[dst_ib.at[b, j]], add=True)

    _drain_scatter()
    plsc.subcore_barrier()

    pltpu.sync_copy(acc_sh.at[pl.ds(sid * _RPT, _RPT)],
                    out_hbm.at[cid, pl.ds(sid * _RPT, _RPT)])
    if want_counts:
        pltpu.sync_copy(cnt_sh.at[pl.ds(sid * _RPT, _RPT)],
                        cnt_hbm.at[cid, pl.ds(sid * _RPT, _RPT)])


def _make_sc_agg(want_counts):
    mesh = plsc.VectorSubcoreMesh(core_axis_name="c", subcore_axis_name="s",
                                  num_cores=_NC, num_subcores=_NS)
    out_type = [jax.ShapeDtypeStruct((_NC, _NACC, _D), jnp.float32)]
    if want_counts:
        out_type.append(jax.ShapeDtypeStruct((_NC, _NACC), jnp.float32))
    scratch = [
        pltpu.VMEM((2, _G, _CHUNK), jnp.int32),
        pltpu.VMEM((2, _G, _CHUNK), jnp.int32),
        pltpu.VMEM((_NB, _CHUNK, _D), jnp.float32),
        pltpu.VMEM((_CHUNK,), jnp.float32),
        pltpu.VMEM_SHARED((_NACC, _D), jnp.float32),
        pltpu.VMEM_SHARED((_NACC,), jnp.float32),
        pltpu.SemaphoreType.DMA,
        pltpu.SemaphoreType.DMA,
        pltpu.SemaphoreType.DMA,
    ]
    return pl.kernel(
        functools.partial(_sc_agg_body, want_counts),
        out_type=out_type if want_counts else out_type[0],
        mesh=mesh,
        scratch_types=scratch,
    )


_BLK = 2000
_GRID = _N // _BLK


def _tc1_body(p_ref, i_ref, x_ref, w1l_ref, b1_ref, w1r_ref, w2l_ref,
              w2r_ref, b2_ref, t2_ref, r2_ref):
    agg = (p_ref[0] + p_ref[1]) * i_ref[...]
    h1 = agg @ w1l_ref[...] + b1_ref[...] + x_ref[...] @ w1r_ref[...]
    h1 = jnp.maximum(h1, 0.0)
    t2_ref[...] = h1 @ w2l_ref[...]
    r2_ref[...] = h1 @ w2r_ref[...] + b2_ref[...]


def _tc1(P, C, xpad, W1l, b1, W1r, W2l, W2r, b2):
    return pl.pallas_call(
        _tc1_body,
        grid=(_GRID,),
        in_specs=[
            pl.BlockSpec((2, _BLK, _D), lambda i: (0, i, 0)),
            pl.BlockSpec((_BLK, 1), lambda i: (i, 0)),
            pl.BlockSpec((_BLK, _D), lambda i: (i, 0)),
            pl.BlockSpec((_D, 256), lambda i: (0, 0)),
            pl.BlockSpec((1, 256), lambda i: (0, 0)),
            pl.BlockSpec((_D, 256), lambda i: (0, 0)),
            pl.BlockSpec((256, _D), lambda i: (0, 0)),
            pl.BlockSpec((256, _D), lambda i: (0, 0)),
            pl.BlockSpec((1, _D), lambda i: (0, 0)),
        ],
        out_specs=[
            pl.BlockSpec((_BLK, _D), lambda i: (i, 0)),
            pl.BlockSpec((_BLK, _D), lambda i: (i, 0)),
        ],
        out_shape=[
            jax.ShapeDtypeStruct((_N, _D), jnp.float32),
            jax.ShapeDtypeStruct((_N, _D), jnp.float32),
        ],
    )(P, C, xpad, W1l, b1, W1r, W2l, W2r, b2)


def _tc2_body(s_ref, i_ref, r_ref, wl_ref, wr_ref, b_ref, t_ref, rout_ref):
    h = (s_ref[0] + s_ref[1]) * i_ref[...] + r_ref[...]
    h = jnp.maximum(h, 0.0)
    t_ref[...] = h @ wl_ref[...]
    rout_ref[...] = h @ wr_ref[...] + b_ref[...]


def _tc2(S, C, r, Wl, Wr, b):
    return pl.pallas_call(
        _tc2_body,
        grid=(_GRID,),
        in_specs=[
            pl.BlockSpec((2, _BLK, _D), lambda i: (0, i, 0)),
            pl.BlockSpec((_BLK, 1), lambda i: (i, 0)),
            pl.BlockSpec((_BLK, _D), lambda i: (i, 0)),
            pl.BlockSpec((_D, _D), lambda i: (0, 0)),
            pl.BlockSpec((_D, _D), lambda i: (0, 0)),
            pl.BlockSpec((1, _D), lambda i: (0, 0)),
        ],
        out_specs=[
            pl.BlockSpec((_BLK, _D), lambda i: (i, 0)),
            pl.BlockSpec((_BLK, _D), lambda i: (i, 0)),
        ],
        out_shape=[
            jax.ShapeDtypeStruct((_N, _D), jnp.float32),
            jax.ShapeDtypeStruct((_N, _D), jnp.float32),
        ],
    )(S, C, r, Wl, Wr, b)


def _tc3_body(s_ref, i_ref, r_ref, o_ref):
    h = (s_ref[0] + s_ref[1]) * i_ref[...] + r_ref[...]
    o_ref[...] = jax.nn.sigmoid(h)


def _tc3(S, C, r):
    return pl.pallas_call(
        _tc3_body,
        grid=(_GRID,),
        in_specs=[
            pl.BlockSpec((2, _BLK, _D), lambda i: (0, i, 0)),
            pl.BlockSpec((_BLK, 1), lambda i: (i, 0)),
            pl.BlockSpec((_BLK, _D), lambda i: (i, 0)),
        ],
        out_specs=pl.BlockSpec((_BLK, _D), lambda i: (i, 0)),
        out_shape=jax.ShapeDtypeStruct((_N, _D), jnp.float32),
    )(S, C, r)


def kernel(x, edge_index, W1l, b1, W1r, W2l, b2, W2r, W3l, b3, W3r):
    npad = _EPAD - _E
    pad_i = jnp.arange(npad, dtype=jnp.int32)
    pads = jnp.stack([(pad_i * 97) % _N, _N + pad_i % (_NACC - _N)])
    edges = jnp.concatenate([edge_index, pads], axis=1)
    edges = edges.reshape(2, _NW, _CPT, _CHUNK)

    S1, Craw = _make_sc_agg(True)(x, edges)
    inv = (1.0 / jnp.maximum(Craw[0, :_N] + Craw[1, :_N], 1.0))[:, None]
    t2, r2 = _tc1(S1, inv, x, W1l, b1.reshape(1, -1), W1r, W2l, W2r,
                  b2.reshape(1, -1))
    S2 = _make_sc_agg(False)(t2, edges)
    t3, r3 = _tc2(S2, inv, r2, W3l, W3r, b3.reshape(1, -1))
    S3 = _make_sc_agg(False)(t3, edges)
    return _tc3(S3, inv, r3)


# speedup vs baseline: 1.2951x; 1.0145x over previous
import functools

import jax
import jax.numpy as jnp
from jax import lax
from jax.experimental import pallas as pl
from jax.experimental.pallas import tpu as pltpu
from jax.experimental.pallas import tpu_sc as plsc

_N = 10000
_E = 320000
_D = 128
_NACC = 10240
_CHUNK = 64
_NC = 2
_NS = 16
_NW = _NC * _NS
_G = 8
_NG = 20
_CPT = _G * _NG
_NB = 4
_RPN = _N // _NS
_EPAD = _NW * _CHUNK * _CPT
_RPT = _NACC // _NS


def _sc_agg_body(want_counts, table_hbm, edges_hbm, *refs):
    if want_counts:
        out_hbm, cnt_hbm = refs[0], refs[1]
        scratch = refs[2:]
    else:
        out_hbm = refs[0]
        scratch = refs[1:]
    (src_ib, dst_ib, rows_v, ones_v, acc_sh, cnt_sh, sem_g, sem_i,
     sem_s) = scratch

    cid = lax.axis_index("c")
    sid = lax.axis_index("s")

    zeros16 = jnp.zeros((16,), jnp.float32)
    ones16 = jnp.ones((16,), jnp.float32)
    for k in range(_CHUNK // 16):
        ones_v[pl.ds(k * 16, 16)] = ones16

    @pl.loop(0, _CHUNK)
    def _(i):
        for k in range(_D // 16):
            rows_v[0, i, pl.ds(k * 16, 16)] = zeros16

    for b in range(_RPT // _CHUNK):
        pltpu.sync_copy(rows_v.at[0],
                        acc_sh.at[pl.ds(sid * _RPT + b * _CHUNK, _CHUNK)])
    if want_counts:
        for b in range(_RPT // _D):
            pltpu.sync_copy(rows_v.at[0, 0],
                            cnt_sh.at[pl.ds(sid * _RPT + b * _D, _D)])
    plsc.subcore_barrier()

    wid = sid * _NC + cid

    def _drain_gather(buf):
        pltpu.make_async_copy(table_hbm.at[src_ib.at[0, 0]], buf, sem_g).wait()

    def _drain_idx():
        pltpu.make_async_copy(edges_hbm.at[0, wid, pl.ds(0, _G // 2)],
                              src_ib.at[0], sem_i).wait()
        pltpu.make_async_copy(edges_hbm.at[1, wid, pl.ds(0, _G // 2)],
                              dst_ib.at[0], sem_i).wait()

    def _drain_scatter():
        pltpu.make_async_copy(rows_v.at[0],
                              acc_sh.at[src_ib.at[0, 0]], sem_s).wait()

    def _sidx(buf, j):
        return buf.at[j // 2, pl.ds((j % 2) * _CHUNK, _CHUNK)]

    pltpu.sync_copy(edges_hbm.at[0, wid, pl.ds(0, _G // 2)], src_ib.at[0])
    pltpu.sync_copy(edges_hbm.at[1, wid, pl.ds(0, _G // 2)], dst_ib.at[0])
    for j in range(_NB - 1):
        pltpu.async_copy(table_hbm.at[_sidx(src_ib.at[0], j)], rows_v.at[j],
                         sem_g)

    @pl.loop(0, _NG)
    def _(g):
        b = g % 2
        for j in range(_G):
            _drain_gather(rows_v.at[j % _NB])
            if j == 0:
                @pl.when(g > 0)
                def _():
                    _drain_scatter()

                @pl.when(g + 1 < _NG)
                def _():
                    pltpu.async_copy(
                        edges_hbm.at[0, wid, pl.ds((g + 1) * (_G // 2), _G // 2)],
                        src_ib.at[(g + 1) % 2], sem_i)
                    pltpu.async_copy(
                        edges_hbm.at[1, wid, pl.ds((g + 1) * (_G // 2), _G // 2)],
                        dst_ib.at[(g + 1) % 2], sem_i)
            else:
                _drain_scatter()
            if j + _NB - 1 < _G:
                pltpu.async_copy(
                    table_hbm.at[_sidx(src_ib.at[b], j + _NB - 1)],
                    rows_v.at[(j + _NB - 1) % _NB], sem_g)
            else:
                if j + _NB - 1 == _G:
                    @pl.when(g + 1 < _NG)
                    def _():
                        _drain_idx()
                        pltpu.async_copy(
                            table_hbm.at[_sidx(src_ib.at[(g + 1) % 2], 0)],
                            rows_v.at[(j + _NB - 1) % _NB], sem_g)
                else:
                    @pl.when(g + 1 < _NG)
                    def _():
                        pltpu.async_copy(
                            table_hbm.at[_sidx(src_ib.at[(g + 1) % 2],
                                               j + _NB - 1 - _G)],
                            rows_v.at[(j + _NB - 1) % _NB], sem_g)
            pltpu.async_copy(rows_v.at[j % _NB],
                             acc_sh.at[_sidx(dst_ib.at[b], j)],
                             sem_s, add=True)
            if want_counts:
                pltpu.sync_copy(ones_v, cnt_sh.at[_sidx(dst_ib.at[b], j)],
                                add=True)

    _drain_scatter()
    plsc.subcore_barrier()

    pltpu.sync_copy(acc_sh.at[pl.ds(sid * _RPT, _RPT)],
                    out_hbm.at[cid, pl.ds(sid * _RPT, _RPT)])
    if want_counts:
        pltpu.sync_copy(cnt_sh.at[pl.ds(sid * _RPT, _RPT)],
                        cnt_hbm.at[cid, pl.ds(sid * _RPT, _RPT)])


def _make_sc_agg(want_counts):
    mesh = plsc.VectorSubcoreMesh(core_axis_name="c", subcore_axis_name="s",
                                  num_cores=_NC, num_subcores=_NS)
    out_type = [jax.ShapeDtypeStruct((_NC, _NACC, _D), jnp.float32)]
    if want_counts:
        out_type.append(jax.ShapeDtypeStruct((_NC, _NACC), jnp.float32))
    scratch = [
        pltpu.VMEM((2, _G // 2, 2 * _CHUNK), jnp.int32),
        pltpu.VMEM((2, _G // 2, 2 * _CHUNK), jnp.int32),
        pltpu.VMEM((_NB, _CHUNK, _D), jnp.float32),
        pltpu.VMEM((_CHUNK,), jnp.float32),
        pltpu.VMEM_SHARED((_NACC, _D), jnp.float32),
        pltpu.VMEM_SHARED((_NACC,), jnp.float32),
        pltpu.SemaphoreType.DMA,
        pltpu.SemaphoreType.DMA,
        pltpu.SemaphoreType.DMA,
    ]
    return pl.kernel(
        functools.partial(_sc_agg_body, want_counts),
        out_type=out_type if want_counts else out_type[0],
        mesh=mesh,
        scratch_types=scratch,
    )


_BLK = 2000
_GRID = _N // _BLK


def _tc1_body(p_ref, i_ref, x_ref, w1l_ref, b1_ref, w1r_ref, w2l_ref,
              w2r_ref, b2_ref, t2_ref, r2_ref):
    agg = (p_ref[0] + p_ref[1]) * i_ref[...]
    h1 = agg @ w1l_ref[...] + b1_ref[...] + x_ref[...] @ w1r_ref[...]
    h1 = jnp.maximum(h1, 0.0)
    t2_ref[...] = h1 @ w2l_ref[...]
    r2_ref[...] = h1 @ w2r_ref[...] + b2_ref[...]


def _tc1(P, C, xpad, W1l, b1, W1r, W2l, W2r, b2):
    return pl.pallas_call(
        _tc1_body,
        grid=(_GRID,),
        in_specs=[
            pl.BlockSpec((2, _BLK, _D), lambda i: (0, i, 0)),
            pl.BlockSpec((_BLK, 1), lambda i: (i, 0)),
            pl.BlockSpec((_BLK, _D), lambda i: (i, 0)),
            pl.BlockSpec((_D, 256), lambda i: (0, 0)),
            pl.BlockSpec((1, 256), lambda i: (0, 0)),
            pl.BlockSpec((_D, 256), lambda i: (0, 0)),
            pl.BlockSpec((256, _D), lambda i: (0, 0)),
            pl.BlockSpec((256, _D), lambda i: (0, 0)),
            pl.BlockSpec((1, _D), lambda i: (0, 0)),
        ],
        out_specs=[
            pl.BlockSpec((_BLK, _D), lambda i: (i, 0)),
            pl.BlockSpec((_BLK, _D), lambda i: (i, 0)),
        ],
        out_shape=[
            jax.ShapeDtypeStruct((_N, _D), jnp.float32),
            jax.ShapeDtypeStruct((_N, _D), jnp.float32),
        ],
    )(P, C, xpad, W1l, b1, W1r, W2l, W2r, b2)


def _tc2_body(s_ref, i_ref, r_ref, wl_ref, wr_ref, b_ref, t_ref, rout_ref):
    h = (s_ref[0] + s_ref[1]) * i_ref[...] + r_ref[...]
    h = jnp.maximum(h, 0.0)
    t_ref[...] = h @ wl_ref[...]
    rout_ref[...] = h @ wr_ref[...] + b_ref[...]


def _tc2(S, C, r, Wl, Wr, b):
    return pl.pallas_call(
        _tc2_body,
        grid=(_GRID,),
        in_specs=[
            pl.BlockSpec((2, _BLK, _D), lambda i: (0, i, 0)),
            pl.BlockSpec((_BLK, 1), lambda i: (i, 0)),
            pl.BlockSpec((_BLK, _D), lambda i: (i, 0)),
            pl.BlockSpec((_D, _D), lambda i: (0, 0)),
            pl.BlockSpec((_D, _D), lambda i: (0, 0)),
            pl.BlockSpec((1, _D), lambda i: (0, 0)),
        ],
        out_specs=[
            pl.BlockSpec((_BLK, _D), lambda i: (i, 0)),
            pl.BlockSpec((_BLK, _D), lambda i: (i, 0)),
        ],
        out_shape=[
            jax.ShapeDtypeStruct((_N, _D), jnp.float32),
            jax.ShapeDtypeStruct((_N, _D), jnp.float32),
        ],
    )(S, C, r, Wl, Wr, b)


def _tc3_body(s_ref, i_ref, r_ref, o_ref):
    h = (s_ref[0] + s_ref[1]) * i_ref[...] + r_ref[...]
    o_ref[...] = jax.nn.sigmoid(h)


def _tc3(S, C, r):
    return pl.pallas_call(
        _tc3_body,
        grid=(_GRID,),
        in_specs=[
            pl.BlockSpec((2, _BLK, _D), lambda i: (0, i, 0)),
            pl.BlockSpec((_BLK, 1), lambda i: (i, 0)),
            pl.BlockSpec((_BLK, _D), lambda i: (i, 0)),
        ],
        out_specs=pl.BlockSpec((_BLK, _D), lambda i: (i, 0)),
        out_shape=jax.ShapeDtypeStruct((_N, _D), jnp.float32),
    )(S, C, r)


def kernel(x, edge_index, W1l, b1, W1r, W2l, b2, W2r, W3l, b3, W3r):
    npad = _EPAD - _E
    pad_i = jnp.arange(npad, dtype=jnp.int32)
    pads = jnp.stack([(pad_i * 97) % _N, _N + pad_i % (_NACC - _N)])
    edges = jnp.concatenate([edge_index, pads], axis=1)
    edges = edges.reshape(2, _NW, _CPT // 2, 2 * _CHUNK)

    S1, Craw = _make_sc_agg(True)(x, edges)
    inv = (1.0 / jnp.maximum(Craw[0, :_N] + Craw[1, :_N], 1.0))[:, None]
    t2, r2 = _tc1(S1, inv, x, W1l, b1.reshape(1, -1), W1r, W2l, W2r,
                  b2.reshape(1, -1))
    S2 = _make_sc_agg(False)(t2, edges)
    t3, r3 = _tc2(S2, inv, r2, W3l, W3r, b3.reshape(1, -1))
    S3 = _make_sc_agg(False)(t3, edges)
    return _tc3(S3, inv, r3)
